# baseline (speedup 1.0000x reference)
import functools

import jax
import jax.numpy as jnp
from jax import lax
from jax.experimental import pallas as pl
from jax.experimental.pallas import tpu as pltpu
from jax.experimental.pallas import tpu_sc as plsc

V = 1_000_000
D = 64
B_ROWS = 16384
B_COLS = 26
NB = B_ROWS * B_COLS
NC = 2
NS = 16
NW = NC * NS
PER_W = NB // NW
CHUNK = 128
NCHUNK = PER_W // CHUNK
RC = 4
RROWS = RC * CHUNK
NR = NCHUNK // RC

VMAIN = 999_936
NWIN = 7813
ST_ROWS = 500_032
VPAD = 2 * ST_ROWS


def _retile(weight_t, wtail_t):
    mesh = plsc.VectorSubcoreMesh(core_axis_name="c", subcore_axis_name="s")

    @functools.partial(
        pl.kernel,
        mesh=mesh,
        out_type=jax.ShapeDtypeStruct((ST_ROWS, 128), jnp.float32),
        scratch_types=[
            pltpu.VMEM((2, 64, 128), jnp.float32),
            pltpu.VMEM((2, 64, 128), jnp.float32),
            pltpu.SemaphoreType.DMA,
            pltpu.SemaphoreType.DMA,
            pltpu.SemaphoreType.DMA,
            pltpu.SemaphoreType.DMA,
        ],
        compiler_params=pltpu.CompilerParams(
            use_tc_tiling_on_sc=True, needs_layout_passes=False),
    )
    def retile(wt_hbm, wtail_hbm, st_hbm, abuf, bbuf, isem0, isem1, osem0, osem1):
        wid = lax.axis_index("s") * NC + lax.axis_index("c")
        nwin = jnp.where(wid < NWIN - 244 * NW, 245, 244)

        def fire_in(k, slot, isem):
            w = wid + k * NW

            @pl.when(w < NWIN - 1)
            def _():
                pltpu.async_copy(
                    wt_hbm.at[:, pl.ds(w * 128, 128)], abuf.at[slot], isem)

            @pl.when(w == NWIN - 1)
            def _():
                pltpu.async_copy(wtail_hbm, abuf.at[slot], isem)

        def wait_in(slot, isem):
            pltpu.make_async_copy(
                wt_hbm.at[:, pl.ds(0, 128)], abuf.at[slot], isem).wait()

        def fire_out(k, slot, osem):
            w = wid + k * NW
            pltpu.async_copy(
                bbuf.at[slot], st_hbm.at[pl.ds(w * 64, 64)], osem)

        def wait_out(slot, osem):
            pltpu.make_async_copy(
                bbuf.at[slot], st_hbm.at[pl.ds(0, 64)], osem).wait()

        rowvs = [lax.iota(jnp.int32, 16) + g * 16 for g in range(4)]

        def transpose(slot):
            src = abuf.at[slot]

            @plsc.parallel_loop(0, 64, unroll=16)
            def _body(p):
                for h in range(2):
                    colv = jnp.full((16,), 2 * p + h, jnp.int32)
                    for g4 in range(4):
                        vals = plsc.load_gather(src, [rowvs[g4], colv])
                        bbuf[slot, p, pl.ds((h * 4 + g4) * 16, 16)] = vals

        fire_in(0, 0, isem0)

        def step(k, slot, isem, osem, isem_next):
            @pl.when(k + 1 < nwin)
            def _():
                fire_in(k + 1, 1 - slot, isem_next)
            wait_in(slot, isem)

            @pl.when(k >= 2)
            def _():
                wait_out(slot, osem)
            transpose(slot)
            fire_out(k, slot, osem)

        def body(i, carry):
            k = 2 * i
            step(k, 0, isem0, osem0, isem1)
            step(k + 1, 1, isem1, osem1, isem0)
            return carry

        lax.fori_loop(0, 122, body, 0)

        @pl.when(nwin == 245)
        def _():
            step(244, 0, isem0, osem0, isem1)
        wait_out(0, osem0)
        wait_out(1, osem1)

    return retile(weight_t, wtail_t)


def _gather(st2, idx):
    mesh = plsc.VectorSubcoreMesh(core_axis_name="c", subcore_axis_name="s")

    @functools.partial(
        pl.kernel,
        mesh=mesh,
        out_type=jax.ShapeDtypeStruct((NB, D), jnp.float32),
        scratch_types=[
            pltpu.VMEM((NCHUNK, CHUNK), jnp.int32),
            pltpu.VMEM((RROWS, D), jnp.float32),
            pltpu.VMEM((RROWS, D), jnp.float32),
            pltpu.SemaphoreType.DMA,
            pltpu.SemaphoreType.DMA,
            pltpu.SemaphoreType.DMA,
            pltpu.SemaphoreType.DMA,
        ],
        compiler_params=pltpu.CompilerParams(use_tc_tiling_on_sc=False),
    )
    def sc_gather(table_hbm, idx_hbm, out_hbm, idx_v,
                  rows0, rows1, gsem0, gsem1, osem0, osem1):
        wid = lax.axis_index("s") * NC + lax.axis_index("c")
        base = wid * PER_W
        pltpu.sync_copy(idx_hbm.at[wid], idx_v)

        def fire_gathers(r, buf, gsem):
            for c in range(RC):
                pltpu.async_copy(
                    table_hbm.at[idx_v.at[r * RC + c]],
                    buf.at[pl.ds(c * CHUNK, CHUNK)], gsem)

        def wait_gathers(buf, gsem):
            for c in range(RC):
                pltpu.make_async_copy(
                    table_hbm.at[pl.ds(0, CHUNK)],
                    buf.at[pl.ds(c * CHUNK, CHUNK)], gsem).wait()

        def out_slice(r):
            return out_hbm.at[pl.ds(base + r * RROWS, RROWS)]

        def half_round(r, bufA, gsemA, osemA, bufB, gsemB, osemB):
            wait_gathers(bufA, gsemA)
            pltpu.async_copy(bufA, out_slice(r), osemA)

            @pl.when(r >= 1)
            def _():
                pltpu.make_async_copy(bufB, out_slice(0), osemB).wait()

            @pl.when(r + 1 < NR)
            def _():
                fire_gathers(r + 1, bufB, gsemB)

        fire_gathers(0, rows0, gsem0)

        def body(k, carry):
            r = 2 * k
            half_round(r, rows0, gsem0, osem0, rows1, gsem1, osem1)
            half_round(r + 1, rows1, gsem1, osem1, rows0, gsem0, osem0)
            return carry

        lax.fori_loop(0, NR // 2, body, 0)
        pltpu.make_async_copy(rows1, out_slice(0), osem1).wait()

    return sc_gather(st2, idx)


def kernel(input_, weight):
    wt = weight.T
    wtail_t = jnp.pad(wt[:, VMAIN:], ((0, 0), (0, 128 - (V - VMAIN))))
    st = _retile(wt, wtail_t)
    st2 = st.reshape(VPAD, D)
    idx = input_.reshape(NW, NCHUNK, CHUNK)
    out = _gather(st2, idx)
    return out.reshape(B_ROWS, B_COLS, D)


# speedup vs baseline: 1.0425x; 1.0425x over previous
import functools

import jax
import jax.numpy as jnp
from jax import lax
from jax.experimental import pallas as pl
from jax.experimental.pallas import tpu as pltpu
from jax.experimental.pallas import tpu_sc as plsc

V = 1_000_000
D = 64
B_ROWS = 16384
B_COLS = 26
NB = B_ROWS * B_COLS
NC = 2
NS = 16
NW = NC * NS
PER_W = NB // NW
CHUNK = 128
NCHUNK = PER_W // CHUNK
RC = 4
RROWS = RC * CHUNK
NR = NCHUNK // RC

VMAIN = 999_936
NWIN = 7813
ST_ROWS = 500_032
VPAD = 2 * ST_ROWS


def _retile(weight_t, wtail_t):
    mesh = plsc.VectorSubcoreMesh(core_axis_name="c", subcore_axis_name="s")

    @functools.partial(
        pl.kernel,
        mesh=mesh,
        out_type=jax.ShapeDtypeStruct((ST_ROWS, 128), jnp.float32),
        scratch_types=[
            pltpu.VMEM((2, 64, 129), jnp.float32),
            pltpu.VMEM((2, 64, 128), jnp.float32),
            pltpu.SemaphoreType.DMA,
            pltpu.SemaphoreType.DMA,
            pltpu.SemaphoreType.DMA,
            pltpu.SemaphoreType.DMA,
        ],
        compiler_params=pltpu.CompilerParams(
            use_tc_tiling_on_sc=True, needs_layout_passes=False),
    )
    def retile(wt_hbm, wtail_hbm, st_hbm, abuf, bbuf, isem0, isem1, osem0, osem1):
        wid = lax.axis_index("s") * NC + lax.axis_index("c")
        nwin = jnp.where(wid < NWIN - 244 * NW, 245, 244)

        def fire_in(k, slot, isem):
            w = wid + k * NW

            @pl.when(w < NWIN - 1)
            def _():
                pltpu.async_copy(
                    wt_hbm.at[:, pl.ds(w * 128, 128)],
                    abuf.at[slot, :, pl.ds(0, 128)], isem)

            @pl.when(w == NWIN - 1)
            def _():
                pltpu.async_copy(wtail_hbm, abuf.at[slot, :, pl.ds(0, 128)], isem)

        def wait_in(slot, isem):
            pltpu.make_async_copy(
                wt_hbm.at[:, pl.ds(0, 128)],
                abuf.at[slot, :, pl.ds(0, 128)], isem).wait()

        def fire_out(k, slot, osem):
            w = wid + k * NW
            pltpu.async_copy(
                bbuf.at[slot], st_hbm.at[pl.ds(w * 64, 64)], osem)

        def wait_out(slot, osem):
            pltpu.make_async_copy(
                bbuf.at[slot], st_hbm.at[pl.ds(0, 64)], osem).wait()

        rowvs = [lax.iota(jnp.int32, 16) + g * 16 for g in range(4)]

        def transpose(slot):
            src = abuf.at[slot]

            @plsc.parallel_loop(0, 64, unroll=8)
            def _body(p):
                for h in range(2):
                    colv = jnp.full((16,), 2 * p + h, jnp.int32)
                    for g4 in range(4):
                        vals = plsc.load_gather(src, [rowvs[g4], colv])
                        bbuf[slot, p, pl.ds((h * 4 + g4) * 16, 16)] = vals

        fire_in(0, 0, isem0)

        def step(k, slot, isem, osem, isem_next):
            @pl.when(k + 1 < nwin)
            def _():
                fire_in(k + 1, 1 - slot, isem_next)
            wait_in(slot, isem)

            @pl.when(k >= 2)
            def _():
                wait_out(slot, osem)
            transpose(slot)
            fire_out(k, slot, osem)

        def body(i, carry):
            k = 2 * i
            step(k, 0, isem0, osem0, isem1)
            step(k + 1, 1, isem1, osem1, isem0)
            return carry

        lax.fori_loop(0, 122, body, 0)

        @pl.when(nwin == 245)
        def _():
            step(244, 0, isem0, osem0, isem1)
        wait_out(0, osem0)
        wait_out(1, osem1)

    return retile(weight_t, wtail_t)


def _gather(st2, idx):
    mesh = plsc.VectorSubcoreMesh(core_axis_name="c", subcore_axis_name="s")

    @functools.partial(
        pl.kernel,
        mesh=mesh,
        out_type=jax.ShapeDtypeStruct((NB, D), jnp.float32),
        scratch_types=[
            pltpu.VMEM((NCHUNK, CHUNK), jnp.int32),
            pltpu.VMEM((RROWS, D), jnp.float32),
            pltpu.VMEM((RROWS, D), jnp.float32),
            pltpu.SemaphoreType.DMA,
            pltpu.SemaphoreType.DMA,
            pltpu.SemaphoreType.DMA,
            pltpu.SemaphoreType.DMA,
        ],
        compiler_params=pltpu.CompilerParams(use_tc_tiling_on_sc=False),
    )
    def sc_gather(table_hbm, idx_hbm, out_hbm, idx_v,
                  rows0, rows1, gsem0, gsem1, osem0, osem1):
        wid = lax.axis_index("s") * NC + lax.axis_index("c")
        base = wid * PER_W
        pltpu.sync_copy(idx_hbm.at[wid], idx_v)

        def fire_gathers(r, buf, gsem):
            for c in range(RC):
                pltpu.async_copy(
                    table_hbm.at[idx_v.at[r * RC + c]],
                    buf.at[pl.ds(c * CHUNK, CHUNK)], gsem)

        def wait_gathers(buf, gsem):
            for c in range(RC):
                pltpu.make_async_copy(
                    table_hbm.at[pl.ds(0, CHUNK)],
                    buf.at[pl.ds(c * CHUNK, CHUNK)], gsem).wait()

        def out_slice(r):
            return out_hbm.at[pl.ds(base + r * RROWS, RROWS)]

        def half_round(r, bufA, gsemA, osemA, bufB, gsemB, osemB):
            wait_gathers(bufA, gsemA)
            pltpu.async_copy(bufA, out_slice(r), osemA)

            @pl.when(r >= 1)
            def _():
                pltpu.make_async_copy(bufB, out_slice(0), osemB).wait()

            @pl.when(r + 1 < NR)
            def _():
                fire_gathers(r + 1, bufB, gsemB)

        fire_gathers(0, rows0, gsem0)

        def body(k, carry):
            r = 2 * k
            half_round(r, rows0, gsem0, osem0, rows1, gsem1, osem1)
            half_round(r + 1, rows1, gsem1, osem1, rows0, gsem0, osem0)
            return carry

        lax.fori_loop(0, NR // 2, body, 0)
        pltpu.make_async_copy(rows1, out_slice(0), osem1).wait()

    return sc_gather(st2, idx)


def kernel(input_, weight):
    wt = weight.T
    wtail_t = jnp.pad(wt[:, VMAIN:], ((0, 0), (0, 128 - (V - VMAIN))))
    st = _retile(wt, wtail_t)
    st2 = st.reshape(VPAD, D)
    idx = input_.reshape(NW, NCHUNK, CHUNK)
    out = _gather(st2, idx)
    return out.reshape(B_ROWS, B_COLS, D)


# speedup vs baseline: 1.0453x; 1.0028x over previous
import functools

import jax
import jax.numpy as jnp
from jax import lax
from jax.experimental import pallas as pl
from jax.experimental.pallas import tpu as pltpu
from jax.experimental.pallas import tpu_sc as plsc

V = 1_000_000
D = 64
B_ROWS = 16384
B_COLS = 26
NB = B_ROWS * B_COLS
NC = 2
NS = 16
NW = NC * NS
PER_W = NB // NW
CHUNK = 128
NCHUNK = PER_W // CHUNK
RC = 4
RROWS = RC * CHUNK
NR = NCHUNK // RC

VMAIN = 999_936
NWIN = 7813
ST_ROWS = 500_032
VPAD = 2 * ST_ROWS


def _retile(weight_t, wtail_t):
    mesh = plsc.VectorSubcoreMesh(core_axis_name="c", subcore_axis_name="s")

    @functools.partial(
        pl.kernel,
        mesh=mesh,
        out_type=jax.ShapeDtypeStruct((ST_ROWS, 128), jnp.float32),
        scratch_types=[
            pltpu.VMEM((2, 64, 129), jnp.float32),
            pltpu.VMEM((2, 64, 128), jnp.float32),
            pltpu.SemaphoreType.DMA,
            pltpu.SemaphoreType.DMA,
            pltpu.SemaphoreType.DMA,
            pltpu.SemaphoreType.DMA,
        ],
        compiler_params=pltpu.CompilerParams(
            use_tc_tiling_on_sc=True, needs_layout_passes=False),
    )
    def retile(wt_hbm, wtail_hbm, st_hbm, abuf, bbuf, isem0, isem1, osem0, osem1):
        wid = lax.axis_index("s") * NC + lax.axis_index("c")
        nwin = jnp.where(wid < NWIN - 244 * NW, 245, 244)

        def fire_in(k, slot, isem):
            w = wid + k * NW

            @pl.when(w < NWIN - 1)
            def _():
                pltpu.async_copy(
                    wt_hbm.at[:, pl.ds(w * 128, 128)],
                    abuf.at[slot, :, pl.ds(0, 128)], isem)

            @pl.when(w == NWIN - 1)
            def _():
                pltpu.async_copy(wtail_hbm, abuf.at[slot, :, pl.ds(0, 128)], isem)

        def wait_in(slot, isem):
            pltpu.make_async_copy(
                wt_hbm.at[:, pl.ds(0, 128)],
                abuf.at[slot, :, pl.ds(0, 128)], isem).wait()

        def fire_out(k, slot, osem):
            w = wid + k * NW
            pltpu.async_copy(
                bbuf.at[slot], st_hbm.at[pl.ds(w * 64, 64)], osem)

        def wait_out(slot, osem):
            pltpu.make_async_copy(
                bbuf.at[slot], st_hbm.at[pl.ds(0, 64)], osem).wait()

        rowvs = [lax.iota(jnp.int32, 16) + g * 16 for g in range(4)]

        def transpose(slot):
            src = abuf.at[slot]

            @plsc.parallel_loop(0, 64, unroll=4)
            def _body(p):
                for h in range(2):
                    colv = jnp.full((16,), 2 * p + h, jnp.int32)
                    for g4 in range(4):
                        vals = plsc.load_gather(src, [rowvs[g4], colv])
                        bbuf[slot, p, pl.ds((h * 4 + g4) * 16, 16)] = vals

        fire_in(0, 0, isem0)

        def step(k, slot, isem, osem, isem_next):
            @pl.when(k + 1 < nwin)
            def _():
                fire_in(k + 1, 1 - slot, isem_next)
            wait_in(slot, isem)

            @pl.when(k >= 2)
            def _():
                wait_out(slot, osem)
            transpose(slot)
            fire_out(k, slot, osem)

        def body(i, carry):
            k = 2 * i
            step(k, 0, isem0, osem0, isem1)
            step(k + 1, 1, isem1, osem1, isem0)
            return carry

        lax.fori_loop(0, 122, body, 0)

        @pl.when(nwin == 245)
        def _():
            step(244, 0, isem0, osem0, isem1)
        wait_out(0, osem0)
        wait_out(1, osem1)

    return retile(weight_t, wtail_t)


def _gather(st2, idx):
    mesh = plsc.VectorSubcoreMesh(core_axis_name="c", subcore_axis_name="s")

    @functools.partial(
        pl.kernel,
        mesh=mesh,
        out_type=jax.ShapeDtypeStruct((NB, D), jnp.float32),
        scratch_types=[
            pltpu.VMEM((NCHUNK, CHUNK), jnp.int32),
            pltpu.VMEM((RROWS, D), jnp.float32),
            pltpu.VMEM((RROWS, D), jnp.float32),
            pltpu.SemaphoreType.DMA,
            pltpu.SemaphoreType.DMA,
            pltpu.SemaphoreType.DMA,
            pltpu.SemaphoreType.DMA,
        ],
        compiler_params=pltpu.CompilerParams(use_tc_tiling_on_sc=False),
    )
    def sc_gather(table_hbm, idx_hbm, out_hbm, idx_v,
                  rows0, rows1, gsem0, gsem1, osem0, osem1):
        wid = lax.axis_index("s") * NC + lax.axis_index("c")
        base = wid * PER_W
        pltpu.sync_copy(idx_hbm.at[wid], idx_v)

        def fire_gathers(r, buf, gsem):
            for c in range(RC):
                pltpu.async_copy(
                    table_hbm.at[idx_v.at[r * RC + c]],
                    buf.at[pl.ds(c * CHUNK, CHUNK)], gsem)

        def wait_gathers(buf, gsem):
            for c in range(RC):
                pltpu.make_async_copy(
                    table_hbm.at[pl.ds(0, CHUNK)],
                    buf.at[pl.ds(c * CHUNK, CHUNK)], gsem).wait()

        def out_slice(r):
            return out_hbm.at[pl.ds(base + r * RROWS, RROWS)]

        def half_round(r, bufA, gsemA, osemA, bufB, gsemB, osemB):
            wait_gathers(bufA, gsemA)
            pltpu.async_copy(bufA, out_slice(r), osemA)

            @pl.when(r >= 1)
            def _():
                pltpu.make_async_copy(bufB, out_slice(0), osemB).wait()

            @pl.when(r + 1 < NR)
            def _():
                fire_gathers(r + 1, bufB, gsemB)

        fire_gathers(0, rows0, gsem0)

        def body(k, carry):
            r = 2 * k
            half_round(r, rows0, gsem0, osem0, rows1, gsem1, osem1)
            half_round(r + 1, rows1, gsem1, osem1, rows0, gsem0, osem0)
            return carry

        lax.fori_loop(0, NR // 2, body, 0)
        pltpu.make_async_copy(rows1, out_slice(0), osem1).wait()

    return sc_gather(st2, idx)


def kernel(input_, weight):
    wt = weight.T
    wtail_t = jnp.pad(wt[:, VMAIN:], ((0, 0), (0, 128 - (V - VMAIN))))
    st = _retile(wt, wtail_t)
    st2 = st.reshape(VPAD, D)
    idx = input_.reshape(NW, NCHUNK, CHUNK)
    out = _gather(st2, idx)
    return out.reshape(B_ROWS, B_COLS, D)


# speedup vs baseline: 1.5841x; 1.5154x over previous
import functools

import jax
import jax.numpy as jnp
from jax import lax
from jax.experimental import pallas as pl
from jax.experimental.pallas import tpu as pltpu
from jax.experimental.pallas import tpu_sc as plsc

V = 1_000_000
D = 64
B_ROWS = 16384
B_COLS = 26
NB = B_ROWS * B_COLS
NC = 2
NS = 16
NW = NC * NS
PER_W = NB // NW
NCHK = 32
CITEMS = 16 * B_COLS


def kernel(input_, weight):
    idx_perm = (
        input_.reshape(NW, NCHK, 16, B_COLS)
        .transpose(0, 1, 3, 2)
        .reshape(NW, PER_W)
    )
    mesh = plsc.VectorSubcoreMesh(core_axis_name="c", subcore_axis_name="s")

    @functools.partial(
        pl.kernel,
        mesh=mesh,
        out_type=jax.ShapeDtypeStruct((B_COLS, 8, 128, 8, 128), jnp.float32),
        scratch_types=[
            pltpu.VMEM((PER_W,), jnp.int32),
            pltpu.VMEM((2, CITEMS, D), jnp.float32),
            pltpu.VMEM((1, B_COLS, 8, 8, 17), jnp.float32),
            pltpu.SemaphoreType.DMA,
            pltpu.SemaphoreType.DMA,
            pltpu.SemaphoreType.DMA,
            pltpu.SemaphoreType.DMA,
        ],
        compiler_params=pltpu.CompilerParams(
            use_tc_tiling_on_sc=False, needs_layout_passes=False),
    )
    def sc_gather(table_hbm, idx_hbm, out_hbm, idx_v, stag, stout,
                  gsem0, gsem1, osem0, osem1):
        wid = lax.axis_index("s") * NC + lax.axis_index("c")
        pltpu.sync_copy(idx_hbm.at[wid], idx_v)
        gsems = (gsem0, gsem1)
        osems = (osem0, osem1)
        pieces = ((0, 128), (128, 128), (256, 128), (384, 32))

        def fire_gather(c, gslot):
            for off, ln in pieces:
                pltpu.async_copy(
                    table_hbm.at[idx_v.at[pl.ds(c * CITEMS + off, ln)]],
                    stag.at[gslot, pl.ds(off, ln)], gsems[gslot])

        def wait_gather(gslot):
            for off, ln in pieces:
                pltpu.make_async_copy(
                    table_hbm.at[pl.ds(0, ln)],
                    stag.at[gslot, pl.ds(off, ln)], gsems[gslot]).wait()

        def out_dst(c):
            ct = 4 * wid + c // 8
            c0 = (c % 8) * 16
            return out_hbm.at[:, :, ct, :, pl.ds(c0, 16)]

        def fire_out(c, oslot):
            pltpu.async_copy(
                stout.at[oslot, :, :, :, pl.ds(0, 16)], out_dst(c),
                osems[oslot])

        def wait_out(oslot):
            pltpu.make_async_copy(
                stout.at[oslot, :, :, :, pl.ds(0, 16)],
                out_hbm.at[:, :, 0, :, pl.ds(0, 16)], osems[oslot]).wait()

        iota = lax.iota(jnp.int32, 16)
        dtvs = [(iota + 16 * k) >> 3 for k in range(4)]
        rv = iota & 7

        def transpose(gslot, oslot):
            dst = stout.at[oslot]

            @plsc.parallel_loop(0, B_COLS, unroll=2)
            def _body(b2):
                b2v = jnp.full((16,), b2, jnp.int32)
                for j in range(16):
                    jv = jnp.full((16,), j, jnp.int32)
                    s = b2 * 16 + j
                    for k in range(4):
                        vals = stag[gslot, s, pl.ds(16 * k, 16)]
                        plsc.store_scatter(dst, [b2v, dtvs[k], rv, jv], vals)

        fire_gather(0, 0)

        def chunk_step(c, gslot, oslot):
            @pl.when(c + 1 < NCHK)
            def _():
                fire_gather(c + 1, 1 - gslot)
            wait_gather(gslot)

            @pl.when(c >= 1)
            def _():
                wait_out(0)
            transpose(gslot, oslot)
            fire_out(c, oslot)

        def body(i, carry):
            c = 2 * i
            chunk_step(c, 0, 0)
            chunk_step(c + 1, 1, 0)
            return carry

        lax.fori_loop(0, NCHK // 2, body, 0)
        wait_out(0)

    out5 = sc_gather(weight, idx_perm)
    return out5.transpose(2, 4, 0, 1, 3).reshape(B_ROWS, B_COLS, D)
